# Initial kernel scaffold; baseline (speedup 1.0000x reference)
import jax
import jax.numpy as jnp
from jax.experimental import pallas as pl


def kernel(x, edge_index, Wl, bl, Wr, gamma, beta):
    raise NotImplementedError("write your pallas kernel here")


# speedup vs baseline: 4.7152x; 4.7152x over previous
import functools

import jax
import jax.numpy as jnp
from jax import lax
from jax.experimental import pallas as pl
from jax.experimental.pallas import tpu as pltpu
from jax.experimental.pallas import tpu_sc as plsc

N = 10000
E = 320000
D = 128
L = 3

NC = 2
NS = 16
NW = NC * NS
NP = 10240
RPW = NP // NS
ZBR = 64

EPW = E // NW
CK = 80
CH = EPW // CK


def _sc_body(h_hbm, src_hbm, dst_hbm, agg_out, cnt_out, src_c, dst_c,
             rows_v, zb_v, ones_v, zbc_v, agg_sh, cnt_sh, sem):
    cid = lax.axis_index("c")
    sid = lax.axis_index("s")
    wid = cid * NS + sid

    def _zrow(r, _):
        for c in range(D // 16):
            zb_v[r, pl.ds(c * 16, 16)] = jnp.zeros((16,), jnp.float32)
        return 0
    lax.fori_loop(0, ZBR, _zrow, 0)

    def _zcnt(k, _):
        zbc_v[pl.ds(k * 16, 16)] = jnp.zeros((16,), jnp.float32)
        return 0
    lax.fori_loop(0, RPW // 16, _zcnt, 0)

    def _ofill(k, _):
        ones_v[pl.ds(k * 16, 16)] = jnp.ones((16,), jnp.float32)
        return 0
    lax.fori_loop(0, CK // 16, _ofill, 0)

    for k in range(RPW // ZBR):
        pltpu.sync_copy(zb_v, agg_sh.at[pl.ds(sid * RPW + k * ZBR, ZBR)])
    pltpu.sync_copy(zbc_v, cnt_sh.at[pl.ds(sid * RPW, RPW)])
    plsc.subcore_barrier()

    def _step(j, _):
        e0 = wid * EPW + j * CK
        pltpu.sync_copy(src_hbm.at[pl.ds(e0, CK)], src_c)
        pltpu.sync_copy(dst_hbm.at[pl.ds(e0, CK)], dst_c)
        pltpu.async_copy(h_hbm.at[src_c], rows_v, sem).wait()
        pltpu.sync_copy(rows_v, agg_sh.at[dst_c], add=True)
        pltpu.sync_copy(ones_v, cnt_sh.at[dst_c], add=True)
        return 0
    lax.fori_loop(0, CH, _step, 0)

    plsc.subcore_barrier()
    pltpu.sync_copy(agg_sh.at[pl.ds(sid * RPW, RPW)],
                    agg_out.at[cid, pl.ds(sid * RPW, RPW)])
    pltpu.sync_copy(cnt_sh.at[pl.ds(sid * RPW, RPW)], zbc_v)
    pltpu.sync_copy(zbc_v, cnt_out.at[pl.ds(cid * NP + sid * RPW, RPW)])


def _make_sc_agg():
    mesh = plsc.VectorSubcoreMesh(core_axis_name="c", subcore_axis_name="s")
    out_type = (jax.ShapeDtypeStruct((NC, NP, D), jnp.float32),
                jax.ShapeDtypeStruct((NC * NP,), jnp.float32))
    scratch = [
        pltpu.VMEM((CK,), jnp.int32),
        pltpu.VMEM((CK,), jnp.int32),
        pltpu.VMEM((CK, D), jnp.float32),
        pltpu.VMEM((ZBR, D), jnp.float32),
        pltpu.VMEM((CK,), jnp.float32),
        pltpu.VMEM((RPW,), jnp.float32),
        pltpu.VMEM_SHARED((NP, D), jnp.float32),
        pltpu.VMEM_SHARED((NP,), jnp.float32),
        pltpu.SemaphoreType.DMA,
    ]
    return pl.kernel(_sc_body, out_type=out_type, mesh=mesh,
                     scratch_types=scratch)


_sc_agg = _make_sc_agg()


def _dense_body(agg2_ref, cnt2_ref, h_ref, Wl_ref, bl_ref, Wr_ref,
                gamma_ref, beta_ref, relu_ref, out_ref):
    agg = agg2_ref[0, :N] + agg2_ref[1, :N]
    cnt = cnt2_ref[0, :N] + cnt2_ref[1, :N]
    mean = agg / jnp.maximum(cnt[:, None], 1.0)
    h = h_ref[...]
    out = (jnp.dot(mean, Wl_ref[...], preferred_element_type=jnp.float32)
           + bl_ref[...][None, :]
           + jnp.dot(h, Wr_ref[...], preferred_element_type=jnp.float32))
    nrm = jnp.sqrt(jnp.sum(out * out, axis=1, keepdims=True))
    out = out / jnp.maximum(nrm, 1e-12)
    mu = jnp.mean(out, axis=0, keepdims=True)
    var = jnp.mean((out - mu) * (out - mu), axis=0, keepdims=True)
    out = (gamma_ref[...][None, :] * (out - mu) / jnp.sqrt(var + 1e-5)
           + beta_ref[...][None, :] + h)
    out = jnp.where(relu_ref[0] > 0.0, jnp.maximum(out, 0.0), out)
    out_ref[...] = out


_dense = pl.pallas_call(
    _dense_body, out_shape=jax.ShapeDtypeStruct((N, D), jnp.float32))


def kernel(x, edge_index, Wl, bl, Wr, gamma, beta):
    src = edge_index[0]
    dst = edge_index[1]
    relu_flags = jnp.arange(L, dtype=jnp.float32)[::-1].reshape(L, 1)

    def _layer(h, xs):
        Wl_i, bl_i, Wr_i, gamma_i, beta_i, relu_i = xs
        agg2, cnt2 = _sc_agg(h, src, dst)
        h = _dense(agg2, cnt2.reshape(NC, NP), h, Wl_i, bl_i, Wr_i,
                   gamma_i, beta_i, relu_i)
        return h, None

    h, _ = lax.scan(_layer, x,
                    (Wl[:L], bl[:L], Wr[:L], gamma[:L], beta[:L], relu_flags))
    return h
